# baseline (speedup 1.0000x reference)
import functools
import math

import jax
import jax.numpy as jnp
from jax import lax
from jax.experimental import pallas as pl
from jax.experimental.pallas import tpu as pltpu
from jax.experimental.pallas import tpu_sc as plsc

_DEPTH = 11
_NLEVELS = _DEPTH + 1
_N_NODES = 2 ** _NLEVELS - 1
_WIDTH = 2048
_TOKENS = 8192
_TILE = 256
_NTILES = _TOKENS // _TILE
_SHALLOW = 8

_NW = 32
_BPW = _TOKENS // _NW
_CHUNK = 16
_NCH = _BPW // _CHUNK


def _gelu(x):
    return 0.5 * x * (1.0 + lax.erf(x * (1.0 / math.sqrt(2.0))))


def _shallow_kernel(x_ref, w_ref, b_ref, acts_ref, n8_ref):
    x = x_ref[...]
    t = x.shape[0]
    L = lax.dot_general(
        x, w_ref[...], (((1,), (1,)), ((), ())),
        precision=lax.Precision.HIGHEST,
        preferred_element_type=jnp.float32)
    bias_row = b_ref[0:1, :]
    ids = lax.broadcasted_iota(jnp.int32, (t, 256), 1)
    lane = lax.broadcasted_iota(jnp.int32, (t, _SHALLOW), 1)
    n = jnp.zeros((t, 1), jnp.int32)
    acts = jnp.zeros((t, _SHALLOW), jnp.float32)
    for d in range(_SHALLOW):
        sel = ids == n
        logit = jnp.sum(jnp.where(sel, L, 0.0), axis=1, keepdims=True)
        logit = logit + jnp.sum(
            jnp.where(sel, jnp.broadcast_to(bias_row, (t, 256)), 0.0),
            axis=1, keepdims=True)
        act = _gelu(logit)
        acts = jnp.where(lane == d, act, acts)
        n = 2 * n + 1 + (logit > 0.0).astype(jnp.int32)
    acts_ref[...] = acts
    n8_ref[...] = n


def _outsh_kernel(n8_ref, acts_sh_ref, wout_ref, out_ref):
    m8 = n8_ref[...] + 1
    t = m8.shape[0]
    cols = lax.broadcasted_iota(jnp.int32, (t, 256), 1)
    a = jnp.zeros((t, 256), jnp.float32)
    for d in range(_SHALLOW):
        nd = (m8 >> (_SHALLOW - d)) - 1
        a = jnp.where(cols == nd, acts_sh_ref[:, d:d + 1], a)
    out_ref[...] = lax.dot_general(
        a.astype(jnp.bfloat16), wout_ref[...],
        (((1,), (0,)), ((), ())),
        preferred_element_type=jnp.float32)


def _sc_gather_body(w_hbm, idx_hbm, rows_hbm, idx_v, buf0, buf1,
                    sem0, sem1):
    wid = lax.axis_index("s") * 2 + lax.axis_index("c")
    base = wid * _BPW
    pltpu.sync_copy(idx_hbm.at[wid], idx_v)
    bufs = (buf0, buf1)
    sems = (sem0, sem1)
    cps = [None, None]
    cps[0] = pltpu.async_copy(w_hbm.at[idx_v.at[0]], buf0, sem0)
    for c in range(_NCH):
        if c + 1 < _NCH:
            cps[(c + 1) % 2] = pltpu.async_copy(
                w_hbm.at[idx_v.at[c + 1]], bufs[(c + 1) % 2],
                sems[(c + 1) % 2])
        cps[c % 2].wait()
        pltpu.sync_copy(bufs[c % 2],
                        rows_hbm.at[pl.ds(base + c * _CHUNK, _CHUNK)])


@functools.cache
def _make_sc_gather():
    return functools.partial(
        pl.kernel,
        mesh=plsc.VectorSubcoreMesh(core_axis_name="c",
                                    subcore_axis_name="s"),
        out_type=jax.ShapeDtypeStruct((_TOKENS, _WIDTH), jnp.float32),
        scratch_types=[
            pltpu.VMEM((_NCH, _CHUNK), jnp.int32),
            pltpu.VMEM((_CHUNK, _WIDTH), jnp.float32),
            pltpu.VMEM((_CHUNK, _WIDTH), jnp.float32),
            pltpu.SemaphoreType.DMA,
            pltpu.SemaphoreType.DMA,
        ],
    )(_sc_gather_body)


def _gather_rows(w, idx):
    return _make_sc_gather()(w, idx)


def _dot_kernel(x_ref, rows_ref, n_ref, b_ref, act_ref, nnext_ref, *,
                level):
    x = x_ref[...]
    r = rows_ref[...]
    t = x.shape[0]
    start = 2 ** level - 1
    size = 2 ** level
    n = n_ref[...]
    local = n - start
    ids = lax.broadcasted_iota(jnp.int32, (t, size), 1)
    sel = ids == local
    brow = b_ref[level:level + 1, :size]
    bsum = jnp.sum(jnp.where(sel, jnp.broadcast_to(brow, (t, size)), 0.0),
                   axis=1, keepdims=True)
    logit = jnp.sum(x * r, axis=1, keepdims=True) + bsum
    act_ref[...] = _gelu(logit)
    nnext_ref[...] = 2 * n + 1 + (logit > 0.0).astype(jnp.int32)


def _axpy_kernel(prev_ref, act_ref, rows_ref, out_ref):
    out_ref[...] = prev_ref[...] + act_ref[...] * rows_ref[...]


def kernel(input, in_weight, in_bias, out_weight):
    orig_shape = input.shape
    x = input.reshape(-1, _WIDTH)

    bias_sh = jnp.zeros((8, 256), jnp.float32)
    bias_sh = bias_sh.at[0, :255].set(in_bias[:255])
    bias_lvl = jnp.zeros((_NLEVELS + 4, _WIDTH), jnp.float32)
    for d in range(_SHALLOW, _NLEVELS):
        row = jnp.zeros((_WIDTH,), jnp.float32)
        row = lax.dynamic_update_slice(
            row, in_bias[2 ** d - 1: 2 ** (d + 1) - 1], (0,))
        bias_lvl = bias_lvl.at[d].set(row)
    wout_sh = out_weight[:256].astype(jnp.bfloat16)

    acts_sh, n8 = pl.pallas_call(
        _shallow_kernel,
        grid=(_NTILES,),
        in_specs=[
            pl.BlockSpec((_TILE, _WIDTH), lambda i: (i, 0)),
            pl.BlockSpec((256, _WIDTH), lambda i: (0, 0)),
            pl.BlockSpec((8, 256), lambda i: (0, 0)),
        ],
        out_specs=[
            pl.BlockSpec((_TILE, _SHALLOW), lambda i: (i, 0)),
            pl.BlockSpec((_TILE, 1), lambda i: (i, 0)),
        ],
        out_shape=[
            jax.ShapeDtypeStruct((_TOKENS, _SHALLOW), jnp.float32),
            jax.ShapeDtypeStruct((_TOKENS, 1), jnp.int32),
        ],
    )(x, in_weight, bias_sh)

    out = pl.pallas_call(
        _outsh_kernel,
        grid=(_NTILES,),
        in_specs=[
            pl.BlockSpec((_TILE, 1), lambda i: (i, 0)),
            pl.BlockSpec((_TILE, _SHALLOW), lambda i: (i, 0)),
            pl.BlockSpec((256, _WIDTH), lambda i: (0, 0)),
        ],
        out_specs=pl.BlockSpec((_TILE, _WIDTH), lambda i: (i, 0)),
        out_shape=jax.ShapeDtypeStruct((_TOKENS, _WIDTH), jnp.float32),
    )(n8, acts_sh, wout_sh)

    n = n8
    for d in range(_SHALLOW, _NLEVELS):
        idx = n.reshape(_NW, _NCH, _CHUNK)
        rows_in = _gather_rows(in_weight, idx)
        rows_out = _gather_rows(out_weight, idx)
        act_d, nnext = pl.pallas_call(
            functools.partial(_dot_kernel, level=d),
            grid=(_NTILES,),
            in_specs=[
                pl.BlockSpec((_TILE, _WIDTH), lambda i: (i, 0)),
                pl.BlockSpec((_TILE, _WIDTH), lambda i: (i, 0)),
                pl.BlockSpec((_TILE, 1), lambda i: (i, 0)),
                pl.BlockSpec((_NLEVELS + 4, _WIDTH), lambda i: (0, 0)),
            ],
            out_specs=[
                pl.BlockSpec((_TILE, 1), lambda i: (i, 0)),
                pl.BlockSpec((_TILE, 1), lambda i: (i, 0)),
            ],
            out_shape=[
                jax.ShapeDtypeStruct((_TOKENS, 1), jnp.float32),
                jax.ShapeDtypeStruct((_TOKENS, 1), jnp.int32),
            ],
        )(x, rows_in, n, bias_lvl)
        out = pl.pallas_call(
            _axpy_kernel,
            grid=(_NTILES,),
            in_specs=[
                pl.BlockSpec((_TILE, _WIDTH), lambda i: (i, 0)),
                pl.BlockSpec((_TILE, 1), lambda i: (i, 0)),
                pl.BlockSpec((_TILE, _WIDTH), lambda i: (i, 0)),
            ],
            out_specs=pl.BlockSpec((_TILE, _WIDTH), lambda i: (i, 0)),
            out_shape=jax.ShapeDtypeStruct((_TOKENS, _WIDTH), jnp.float32),
            input_output_aliases={0: 0},
        )(out, act_d, rows_out)
        n = nnext

    return out.reshape(orig_shape)


# speedup vs baseline: 1.1146x; 1.1146x over previous
import functools
import math

import jax
import jax.numpy as jnp
import numpy as np
from jax import lax
from jax.experimental import pallas as pl
from jax.experimental.pallas import tpu as pltpu
from jax.experimental.pallas import tpu_sc as plsc

_DEPTH = 11
_NLEVELS = _DEPTH + 1
_N_NODES = 2 ** _NLEVELS - 1
_WIDTH = 2048
_TOKENS = 8192
_TILE = 256
_NTILES = _TOKENS // _TILE
_SHALLOW = 8

_NW = 32
_BPW = _TOKENS // _NW
_CHUNK = 16
_NCH = _BPW // _CHUNK


def _gelu(x):
    return 0.5 * x * (1.0 + lax.erf(x * (1.0 / math.sqrt(2.0))))


def _shallow_kernel(x_ref, w0_ref, w1_ref, w2_ref, b_ref, acts_ref,
                    n8_ref):
    x = x_ref[...]
    t = x.shape[0]
    bias_row = b_ref[0:1, :]
    ids = lax.broadcasted_iota(jnp.int32, (t, 256), 1)
    lane = lax.broadcasted_iota(jnp.int32, (t, _SHALLOW), 1)
    n = jnp.zeros((t, 1), jnp.int32)
    acts = jnp.zeros((t, _SHALLOW), jnp.float32)
    dn = (((1,), (0,)), ((), ()))
    for d in range(_SHALLOW):
        sel = ids == n
        oh = sel.astype(jnp.bfloat16)
        rows = (lax.dot_general(oh, w0_ref[...], dn,
                                preferred_element_type=jnp.float32)
                + lax.dot_general(oh, w1_ref[...], dn,
                                  preferred_element_type=jnp.float32)
                + lax.dot_general(oh, w2_ref[...], dn,
                                  preferred_element_type=jnp.float32))
        bsum = jnp.sum(
            jnp.where(sel, jnp.broadcast_to(bias_row, (t, 256)), 0.0),
            axis=1, keepdims=True)
        logit = jnp.sum(x * rows, axis=1, keepdims=True) + bsum
        act = _gelu(logit)
        acts = jnp.where(lane == d, act, acts)
        n = 2 * n + 1 + (logit > 0.0).astype(jnp.int32)
    acts_ref[...] = acts
    n8_ref[...] = n


def _sc_gather_body(w_hbm, idx_hbm, rows_hbm, idx_v, buf0, buf1, buf2,
                    gs0, gs1, gs2, ws0, ws1, ws2):
    wid = lax.axis_index("s") * 2 + lax.axis_index("c")
    base = wid * _BPW
    pltpu.sync_copy(idx_hbm.at[wid], idx_v)
    bufs = (buf0, buf1, buf2)
    gsems = (gs0, gs1, gs2)
    wsems = (ws0, ws1, ws2)
    g = [None, None, None]
    g[0] = pltpu.async_copy(w_hbm.at[idx_v.at[0]], bufs[0], gsems[0])
    g[1] = pltpu.async_copy(w_hbm.at[idx_v.at[1]], bufs[1], gsems[1])
    for c in range(_NCH):
        b = c % 3
        g[b].wait()
        if c + 2 < _NCH:
            g[(c + 2) % 3] = pltpu.async_copy(
                w_hbm.at[idx_v.at[c + 2]], bufs[(c + 2) % 3],
                gsems[(c + 2) % 3])
        pltpu.sync_copy(bufs[b],
                        rows_hbm.at[pl.ds(base + c * _CHUNK, _CHUNK)])
    del wsems


@functools.cache
def _make_sc_gather():
    return functools.partial(
        pl.kernel,
        mesh=plsc.VectorSubcoreMesh(core_axis_name="c",
                                    subcore_axis_name="s"),
        out_type=jax.ShapeDtypeStruct((_TOKENS, _WIDTH), jnp.float32),
        scratch_types=[
            pltpu.VMEM((_NCH, _CHUNK), jnp.int32),
            pltpu.VMEM((_CHUNK, _WIDTH), jnp.float32),
            pltpu.VMEM((_CHUNK, _WIDTH), jnp.float32),
            pltpu.VMEM((_CHUNK, _WIDTH), jnp.float32),
            pltpu.SemaphoreType.DMA,
            pltpu.SemaphoreType.DMA,
            pltpu.SemaphoreType.DMA,
            pltpu.SemaphoreType.DMA,
            pltpu.SemaphoreType.DMA,
            pltpu.SemaphoreType.DMA,
        ],
    )(_sc_gather_body)


def _gather_rows(w, idx):
    return _make_sc_gather()(w, idx)


def _dot_kernel(x_ref, rows_ref, n_ref, b_ref, act_ref, nnext_ref, *,
                level):
    x = x_ref[...]
    r = rows_ref[...]
    t = x.shape[0]
    start = 2 ** level - 1
    size = 2 ** level
    n = n_ref[...]
    local = n - start
    ids = lax.broadcasted_iota(jnp.int32, (t, size), 1)
    sel = ids == local
    brow = b_ref[level:level + 1, :size]
    bsum = jnp.sum(jnp.where(sel, jnp.broadcast_to(brow, (t, size)), 0.0),
                   axis=1, keepdims=True)
    logit = jnp.sum(x * r, axis=1, keepdims=True) + bsum
    act_ref[...] = _gelu(logit)
    nnext_ref[...] = 2 * n + 1 + (logit > 0.0).astype(jnp.int32)


_COL_LEVEL = np.zeros((_N_NODES + 1,), np.int32)
for _d in range(_NLEVELS):
    _COL_LEVEL[2 ** _d - 1: 2 ** (_d + 1) - 1] = _d
_COL_LEVEL[_N_NODES] = 15
_SHIFT_TAB = np.zeros((8, _N_NODES + 1), np.int32)
_SHIFT_TAB[0, :] = _DEPTH - _COL_LEVEL
_SHIFT_TAB[0, _N_NODES] = 0
_LVLMAP = np.zeros((16, _N_NODES + 1), np.float32)
_LVLMAP[_COL_LEVEL, np.arange(_N_NODES + 1)] = 1.0
_LVLMAP[15, _N_NODES] = 0.0


def _out_kernel(leaf_ref, acts_ref, shift_ref, lvlmap_ref, wout_ref,
                out_ref):
    leaf1 = leaf_ref[...] + 1
    t = leaf1.shape[0]
    cols1 = lax.broadcasted_iota(jnp.int32, (t, _N_NODES + 1), 1) + 1
    shift = shift_ref[0:1, :]
    cond = jnp.right_shift(leaf1, shift) == cols1
    actsel = lax.dot_general(
        acts_ref[...].astype(jnp.bfloat16), lvlmap_ref[...],
        (((1,), (0,)), ((), ())),
        preferred_element_type=jnp.float32)
    a = jnp.where(cond, actsel, 0.0).astype(jnp.bfloat16)
    out_ref[...] = lax.dot_general(
        a, wout_ref[...],
        (((1,), (0,)), ((), ())),
        preferred_element_type=jnp.float32)


def kernel(input, in_weight, in_bias, out_weight):
    orig_shape = input.shape
    x = input.reshape(-1, _WIDTH)

    bias_sh = jnp.zeros((8, 256), jnp.float32)
    bias_sh = bias_sh.at[0, :255].set(in_bias[:255])
    wsh = in_weight[:256]
    mask = jnp.int32(-65536)

    def _trunc(v):
        bits = lax.bitcast_convert_type(v, jnp.int32)
        return lax.bitcast_convert_type(bits & mask, jnp.float32)

    w0f = _trunc(wsh)
    r1 = wsh - w0f
    w1f = _trunc(r1)
    r2 = r1 - w1f
    w0 = w0f.astype(jnp.bfloat16)
    w1 = w1f.astype(jnp.bfloat16)
    w2 = r2.astype(jnp.bfloat16)
    bias_lvl = jnp.zeros((_NLEVELS + 4, _WIDTH), jnp.float32)
    for d in range(_SHALLOW, _NLEVELS):
        row = jnp.zeros((_WIDTH,), jnp.float32)
        row = lax.dynamic_update_slice(
            row, in_bias[2 ** d - 1: 2 ** (d + 1) - 1], (0,))
        bias_lvl = bias_lvl.at[d].set(row)
    wout_p = jnp.pad(out_weight.astype(jnp.bfloat16), ((0, 1), (0, 0)))

    acts_sh, n8 = pl.pallas_call(
        _shallow_kernel,
        grid=(_NTILES,),
        in_specs=[
            pl.BlockSpec((_TILE, _WIDTH), lambda i: (i, 0)),
            pl.BlockSpec((256, _WIDTH), lambda i: (0, 0)),
            pl.BlockSpec((256, _WIDTH), lambda i: (0, 0)),
            pl.BlockSpec((256, _WIDTH), lambda i: (0, 0)),
            pl.BlockSpec((8, 256), lambda i: (0, 0)),
        ],
        out_specs=[
            pl.BlockSpec((_TILE, _SHALLOW), lambda i: (i, 0)),
            pl.BlockSpec((_TILE, 1), lambda i: (i, 0)),
        ],
        out_shape=[
            jax.ShapeDtypeStruct((_TOKENS, _SHALLOW), jnp.float32),
            jax.ShapeDtypeStruct((_TOKENS, 1), jnp.int32),
        ],
    )(x, w0, w1, w2, bias_sh)

    n = n8
    acts_deep = []
    leaf = None
    for d in range(_SHALLOW, _NLEVELS):
        idx = n.reshape(_NW, _NCH, _CHUNK)
        rows = _gather_rows(in_weight, idx)
        if d == _DEPTH:
            leaf = n
        act_d, nnext = pl.pallas_call(
            functools.partial(_dot_kernel, level=d),
            grid=(_NTILES,),
            in_specs=[
                pl.BlockSpec((_TILE, _WIDTH), lambda i: (i, 0)),
                pl.BlockSpec((_TILE, _WIDTH), lambda i: (i, 0)),
                pl.BlockSpec((_TILE, 1), lambda i: (i, 0)),
                pl.BlockSpec((_NLEVELS + 4, _WIDTH), lambda i: (0, 0)),
            ],
            out_specs=[
                pl.BlockSpec((_TILE, 1), lambda i: (i, 0)),
                pl.BlockSpec((_TILE, 1), lambda i: (i, 0)),
            ],
            out_shape=[
                jax.ShapeDtypeStruct((_TOKENS, 1), jnp.float32),
                jax.ShapeDtypeStruct((_TOKENS, 1), jnp.int32),
            ],
        )(x, rows, n, bias_lvl)
        acts_deep.append(act_d)
        n = nnext

    acts16 = jnp.concatenate(
        [acts_sh] + acts_deep + [jnp.zeros((_TOKENS, 4), jnp.float32)],
        axis=1)
    shift_tab = jnp.asarray(_SHIFT_TAB)
    lvlmap = jnp.asarray(_LVLMAP).astype(jnp.bfloat16)
    out = pl.pallas_call(
        _out_kernel,
        grid=(_NTILES,),
        in_specs=[
            pl.BlockSpec((_TILE, 1), lambda i: (i, 0)),
            pl.BlockSpec((_TILE, 16), lambda i: (i, 0)),
            pl.BlockSpec((8, _N_NODES + 1), lambda i: (0, 0)),
            pl.BlockSpec((16, _N_NODES + 1), lambda i: (0, 0)),
            pl.BlockSpec((_N_NODES + 1, _WIDTH), lambda i: (0, 0)),
        ],
        out_specs=pl.BlockSpec((_TILE, _WIDTH), lambda i: (i, 0)),
        out_shape=jax.ShapeDtypeStruct((_TOKENS, _WIDTH), jnp.float32),
    )(leaf, acts16, shift_tab, lvlmap, wout_p)
    return out.reshape(orig_shape)


# speedup vs baseline: 1.2827x; 1.1508x over previous
import functools
import math

import jax
import jax.numpy as jnp
import numpy as np
from jax import lax
from jax.experimental import pallas as pl
from jax.experimental.pallas import tpu as pltpu
from jax.experimental.pallas import tpu_sc as plsc

_DEPTH = 11
_NLEVELS = _DEPTH + 1
_N_NODES = 2 ** _NLEVELS - 1
_WIDTH = 2048
_TOKENS = 8192
_TILE = 256
_NTILES = _TOKENS // _TILE
_SHALLOW = 10

_NW = 32
_BPW = _TOKENS // _NW
_CHUNK = 16
_NCH = _BPW // _CHUNK


def _gelu(x):
    return 0.5 * x * (1.0 + lax.erf(x * (1.0 / math.sqrt(2.0))))


def _walk_level(x, n, sel, wsplit_refs, bias_ref):
    dn = (((1,), (0,)), ((), ()))
    oh = sel.astype(jnp.bfloat16)
    w0_ref, w1_ref, w2_ref = wsplit_refs
    rows = (lax.dot_general(oh, w0_ref[...], dn,
                            preferred_element_type=jnp.float32)
            + lax.dot_general(oh, w1_ref[...], dn,
                              preferred_element_type=jnp.float32)
            + lax.dot_general(oh, w2_ref[...], dn,
                              preferred_element_type=jnp.float32))
    k = sel.shape[1]
    bias_row = bias_ref[0:1, :]
    bsum = jnp.sum(
        jnp.where(sel, jnp.broadcast_to(bias_row, (sel.shape[0], k)), 0.0),
        axis=1, keepdims=True)
    logit = jnp.sum(x * rows, axis=1, keepdims=True) + bsum
    return logit


def _shallow_kernel(x_ref, a0_ref, a1_ref, a2_ref, b0_ref, b1_ref,
                    b2_ref, c0_ref, c1_ref, c2_ref, ba_ref, bb_ref,
                    bc_ref, acts_ref, n10_ref):
    x = x_ref[...]
    t = x.shape[0]
    ids256 = lax.broadcasted_iota(jnp.int32, (t, 256), 1)
    ids512 = lax.broadcasted_iota(jnp.int32, (t, 512), 1)
    lane = lax.broadcasted_iota(jnp.int32, (t, 16), 1)
    n = jnp.zeros((t, 1), jnp.int32)
    acts = jnp.zeros((t, 16), jnp.float32)
    for d in range(_SHALLOW):
        if d < 8:
            sel = ids256 == n
            logit = _walk_level(x, n, sel, (a0_ref, a1_ref, a2_ref),
                                ba_ref)
        elif d == 8:
            sel = ids256 == n - 255
            logit = _walk_level(x, n, sel, (b0_ref, b1_ref, b2_ref),
                                bb_ref)
        else:
            sel = ids512 == n - 511
            logit = _walk_level(x, n, sel, (c0_ref, c1_ref, c2_ref),
                                bc_ref)
        act = _gelu(logit)
        acts = jnp.where(lane == d, act, acts)
        n = 2 * n + 1 + (logit > 0.0).astype(jnp.int32)
    acts_ref[...] = acts
    n10_ref[...] = n


def _sc_gather_body(w_hbm, idx_hbm, rows_hbm, idx_v, buf0, buf1, buf2,
                    gs0, gs1, gs2, ws0, ws1, ws2):
    wid = lax.axis_index("s") * 2 + lax.axis_index("c")
    base = wid * _BPW
    pltpu.sync_copy(idx_hbm.at[wid], idx_v)
    bufs = (buf0, buf1, buf2)
    gsems = (gs0, gs1, gs2)
    wsems = (ws0, ws1, ws2)
    g = [None, None, None]
    g[0] = pltpu.async_copy(w_hbm.at[idx_v.at[0]], bufs[0], gsems[0])
    g[1] = pltpu.async_copy(w_hbm.at[idx_v.at[1]], bufs[1], gsems[1])
    for c in range(_NCH):
        b = c % 3
        g[b].wait()
        if c + 2 < _NCH:
            g[(c + 2) % 3] = pltpu.async_copy(
                w_hbm.at[idx_v.at[c + 2]], bufs[(c + 2) % 3],
                gsems[(c + 2) % 3])
        pltpu.sync_copy(bufs[b],
                        rows_hbm.at[pl.ds(base + c * _CHUNK, _CHUNK)])
    del wsems


@functools.cache
def _make_sc_gather():
    return functools.partial(
        pl.kernel,
        mesh=plsc.VectorSubcoreMesh(core_axis_name="c",
                                    subcore_axis_name="s"),
        out_type=jax.ShapeDtypeStruct((_TOKENS, _WIDTH), jnp.float32),
        scratch_types=[
            pltpu.VMEM((_NCH, _CHUNK), jnp.int32),
            pltpu.VMEM((_CHUNK, _WIDTH), jnp.float32),
            pltpu.VMEM((_CHUNK, _WIDTH), jnp.float32),
            pltpu.VMEM((_CHUNK, _WIDTH), jnp.float32),
            pltpu.SemaphoreType.DMA,
            pltpu.SemaphoreType.DMA,
            pltpu.SemaphoreType.DMA,
            pltpu.SemaphoreType.DMA,
            pltpu.SemaphoreType.DMA,
            pltpu.SemaphoreType.DMA,
        ],
    )(_sc_gather_body)


def _gather_rows(w, idx):
    return _make_sc_gather()(w, idx)


def _dot_kernel(x_ref, rows_ref, n_ref, b_ref, act_ref, nnext_ref, *,
                level):
    x = x_ref[...]
    r = rows_ref[...]
    t = x.shape[0]
    start = 2 ** level - 1
    size = 2 ** level
    n = n_ref[...]
    local = n - start
    ids = lax.broadcasted_iota(jnp.int32, (t, size), 1)
    sel = ids == local
    brow = b_ref[level:level + 1, :size]
    bsum = jnp.sum(jnp.where(sel, jnp.broadcast_to(brow, (t, size)), 0.0),
                   axis=1, keepdims=True)
    logit = jnp.sum(x * r, axis=1, keepdims=True) + bsum
    act_ref[...] = _gelu(logit)
    nnext_ref[...] = 2 * n + 1 + (logit > 0.0).astype(jnp.int32)


_COL_LEVEL = np.zeros((_N_NODES + 1,), np.int32)
for _d in range(_NLEVELS):
    _COL_LEVEL[2 ** _d - 1: 2 ** (_d + 1) - 1] = _d
_COL_LEVEL[_N_NODES] = 15
_SHIFT_TAB = np.zeros((8, _N_NODES + 1), np.int32)
_SHIFT_TAB[0, :] = _DEPTH - _COL_LEVEL
_SHIFT_TAB[0, _N_NODES] = 0
_LVLMAP = np.zeros((16, _N_NODES + 1), np.float32)
_LVLMAP[_COL_LEVEL, np.arange(_N_NODES + 1)] = 1.0
_LVLMAP[15, _N_NODES] = 0.0


def _out_kernel(leaf_ref, acts_ref, shift_ref, lvlmap_ref, wout_ref,
                out_ref):
    leaf1 = leaf_ref[...] + 1
    t = leaf1.shape[0]
    cols1 = lax.broadcasted_iota(jnp.int32, (t, _N_NODES + 1), 1) + 1
    shift = shift_ref[0:1, :]
    cond = jnp.right_shift(leaf1, shift) == cols1
    actsel = lax.dot_general(
        acts_ref[...].astype(jnp.bfloat16), lvlmap_ref[...],
        (((1,), (0,)), ((), ())),
        preferred_element_type=jnp.float32)
    a = jnp.where(cond, actsel, 0.0).astype(jnp.bfloat16)
    out_ref[...] = lax.dot_general(
        a, wout_ref[...],
        (((1,), (0,)), ((), ())),
        preferred_element_type=jnp.float32)


def kernel(input, in_weight, in_bias, out_weight):
    orig_shape = input.shape
    x = input.reshape(-1, _WIDTH)

    mask = jnp.int32(-65536)

    def _trunc(v):
        bits = lax.bitcast_convert_type(v, jnp.int32)
        return lax.bitcast_convert_type(bits & mask, jnp.float32)

    def _split3(w):
        w0f = _trunc(w)
        r1 = w - w0f
        w1f = _trunc(r1)
        r2 = r1 - w1f
        return (w0f.astype(jnp.bfloat16), w1f.astype(jnp.bfloat16),
                r2.astype(jnp.bfloat16))

    ta = _split3(in_weight[:256])
    tb = _split3(in_weight[255:511])
    tc = _split3(in_weight[511:1023])
    bias_a = jnp.zeros((8, 256), jnp.float32)
    bias_a = bias_a.at[0, :255].set(in_bias[:255])
    bias_b = jnp.zeros((8, 256), jnp.float32)
    bias_b = bias_b.at[0, :].set(in_bias[255:511])
    bias_c = jnp.zeros((8, 512), jnp.float32)
    bias_c = bias_c.at[0, :].set(in_bias[511:1023])
    bias_lvl = jnp.zeros((_NLEVELS + 4, _WIDTH), jnp.float32)
    for d in range(_SHALLOW, _NLEVELS):
        row = jnp.zeros((_WIDTH,), jnp.float32)
        row = lax.dynamic_update_slice(
            row, in_bias[2 ** d - 1: 2 ** (d + 1) - 1], (0,))
        bias_lvl = bias_lvl.at[d].set(row)
    wout_p = jnp.pad(out_weight.astype(jnp.bfloat16), ((0, 1), (0, 0)))

    acts_sh, n10 = pl.pallas_call(
        _shallow_kernel,
        grid=(_NTILES,),
        in_specs=[
            pl.BlockSpec((_TILE, _WIDTH), lambda i: (i, 0)),
            pl.BlockSpec((256, _WIDTH), lambda i: (0, 0)),
            pl.BlockSpec((256, _WIDTH), lambda i: (0, 0)),
            pl.BlockSpec((256, _WIDTH), lambda i: (0, 0)),
            pl.BlockSpec((256, _WIDTH), lambda i: (0, 0)),
            pl.BlockSpec((256, _WIDTH), lambda i: (0, 0)),
            pl.BlockSpec((256, _WIDTH), lambda i: (0, 0)),
            pl.BlockSpec((512, _WIDTH), lambda i: (0, 0)),
            pl.BlockSpec((512, _WIDTH), lambda i: (0, 0)),
            pl.BlockSpec((512, _WIDTH), lambda i: (0, 0)),
            pl.BlockSpec((8, 256), lambda i: (0, 0)),
            pl.BlockSpec((8, 256), lambda i: (0, 0)),
            pl.BlockSpec((8, 512), lambda i: (0, 0)),
        ],
        out_specs=[
            pl.BlockSpec((_TILE, 16), lambda i: (i, 0)),
            pl.BlockSpec((_TILE, 1), lambda i: (i, 0)),
        ],
        out_shape=[
            jax.ShapeDtypeStruct((_TOKENS, 16), jnp.float32),
            jax.ShapeDtypeStruct((_TOKENS, 1), jnp.int32),
        ],
    )(x, *ta, *tb, *tc, bias_a, bias_b, bias_c)

    n = n10
    acts_deep = []
    leaf = None
    for d in range(_SHALLOW, _NLEVELS):
        idx = n.reshape(_NW, _NCH, _CHUNK)
        rows = _gather_rows(in_weight, idx)
        if d == _DEPTH:
            leaf = n
        act_d, nnext = pl.pallas_call(
            functools.partial(_dot_kernel, level=d),
            grid=(_NTILES,),
            in_specs=[
                pl.BlockSpec((_TILE, _WIDTH), lambda i: (i, 0)),
                pl.BlockSpec((_TILE, _WIDTH), lambda i: (i, 0)),
                pl.BlockSpec((_TILE, 1), lambda i: (i, 0)),
                pl.BlockSpec((_NLEVELS + 4, _WIDTH), lambda i: (0, 0)),
            ],
            out_specs=[
                pl.BlockSpec((_TILE, 1), lambda i: (i, 0)),
                pl.BlockSpec((_TILE, 1), lambda i: (i, 0)),
            ],
            out_shape=[
                jax.ShapeDtypeStruct((_TOKENS, 1), jnp.float32),
                jax.ShapeDtypeStruct((_TOKENS, 1), jnp.int32),
            ],
        )(x, rows, n, bias_lvl)
        acts_deep.append(act_d)
        n = nnext

    acts16 = jnp.concatenate(
        [lax.slice(acts_sh, (0, 0), (_TOKENS, _SHALLOW))] + acts_deep
        + [jnp.zeros((_TOKENS, 16 - _NLEVELS), jnp.float32)],
        axis=1)
    shift_tab = jnp.asarray(_SHIFT_TAB)
    lvlmap = jnp.asarray(_LVLMAP).astype(jnp.bfloat16)
    out = pl.pallas_call(
        _out_kernel,
        grid=(_NTILES,),
        in_specs=[
            pl.BlockSpec((_TILE, 1), lambda i: (i, 0)),
            pl.BlockSpec((_TILE, 16), lambda i: (i, 0)),
            pl.BlockSpec((8, _N_NODES + 1), lambda i: (0, 0)),
            pl.BlockSpec((16, _N_NODES + 1), lambda i: (0, 0)),
            pl.BlockSpec((_N_NODES + 1, _WIDTH), lambda i: (0, 0)),
        ],
        out_specs=pl.BlockSpec((_TILE, _WIDTH), lambda i: (i, 0)),
        out_shape=jax.ShapeDtypeStruct((_TOKENS, _WIDTH), jnp.float32),
    )(leaf, acts16, shift_tab, lvlmap, wout_p)
    return out.reshape(orig_shape)
